# baseline (speedup 1.0000x reference)
import functools
import math

import jax
import jax.numpy as jnp
from jax import lax
from jax.experimental import pallas as pl
from jax.experimental.pallas import tpu as pltpu
from jax.experimental.pallas import tpu_sc as plsc

N = 10000
E = 320000
D = 128
ED = 16
L = 3
G = 128
STEPS = 2

_NCORE = 2
_NSUB = 16
_NW = _NCORE * _NSUB
_CHUNK = 128
_RPW = 320
_NACC = _RPW * _NW
_EP = 327680

_BN = 1.0 / math.sqrt(1.0 + 1e-5)


def _leaky(v, s=0.01):
    return jnp.where(v >= 0, v, s * v)


_EE_BLK = 8192


def _ee_body(ea_ref, w_ref, b_ref, out_ref):
    out_ref[0] = (
        jnp.dot(ea_ref[...], w_ref[0], preferred_element_type=jnp.float32)
        + b_ref[0]
    )


def _edge_emb(edge_attr_pad, We, be3):
    return pl.pallas_call(
        _ee_body,
        grid=(L, _EP // _EE_BLK),
        in_specs=[
            pl.BlockSpec((_EE_BLK, ED), lambda l, i: (i, 0)),
            pl.BlockSpec((1, ED, D), lambda l, i: (l, 0, 0)),
            pl.BlockSpec((1, 1, D), lambda l, i: (l, 0, 0)),
        ],
        out_specs=pl.BlockSpec((1, _EE_BLK, D), lambda l, i: (l, i, 0)),
        out_shape=jax.ShapeDtypeStruct((L, _EP, D), jnp.float32),
    )(edge_attr_pad, We, be3)


def _sc_body(l, x_hbm, ee_hbm, src_hbm, dstloc_hbm, starts_hbm, out_hbm,
             stv, srcv, dstv, buf, acc, sem_lin, sem_gat):
    c = lax.axis_index("c")
    s = lax.axis_index("s")
    w = c * _NSUB + s

    pltpu.sync_copy(starts_hbm, stv)
    win = stv[pl.ds(w, 16)]
    start = win[0]
    end = win[1]
    abase = (start // 8) * 8
    nch = (end - abase + _CHUNK - 1) // _CHUNK

    zero16 = jnp.zeros((16,), jnp.float32)

    def zrow(r, carry):
        for k in range(8):
            acc[r, pl.ds(k * 16, 16)] = zero16
        return carry

    lax.fori_loop(0, _RPW, zrow, 0)

    def lin_start(cb, b):
        pltpu.async_copy(src_hbm.at[pl.ds(cb, _CHUNK)],
                         srcv.at[b], sem_lin.at[b])
        pltpu.async_copy(dstloc_hbm.at[pl.ds(cb, 2 * _CHUNK)],
                         dstv.at[b, 0], sem_lin.at[b])
        pltpu.async_copy(ee_hbm.at[l, pl.ds(cb, _CHUNK)],
                         buf.at[b], sem_lin.at[b])

    def lin_wait(cb, b):
        pltpu.make_async_copy(src_hbm.at[pl.ds(cb, _CHUNK)],
                              srcv.at[b], sem_lin.at[b]).wait()
        pltpu.make_async_copy(dstloc_hbm.at[pl.ds(cb, 2 * _CHUNK)],
                              dstv.at[b, 0], sem_lin.at[b]).wait()
        pltpu.make_async_copy(ee_hbm.at[l, pl.ds(cb, _CHUNK)],
                              buf.at[b], sem_lin.at[b]).wait()

    def gat_start(b):
        pltpu.async_copy(x_hbm.at[srcv.at[b]], buf.at[b], sem_gat.at[b],
                         add=True)

    def gat_wait(b):
        pltpu.make_async_copy(x_hbm.at[srcv.at[b]], buf.at[b],
                              sem_gat.at[b]).wait()

    def accum(b, cb):
        e_lo = jnp.maximum(start - cb, 0)
        e_hi = jnp.minimum(end - cb, _CHUNK)

        def per_edge(e, carry):
            wv = dstv[b, 0, pl.ds(e, 16)]
            row = wv[0]
            for k in range(8):
                val = jnp.maximum(buf[b, e, pl.ds(k * 16, 16)], 0.0)
                plsc.addupdate(acc.at[row, pl.ds(k * 16, 16)], val)
            return carry

        lax.fori_loop(e_lo, e_hi, per_edge, 0)

    def steps(to, carry):
        for j in range(3):
            t = 3 * to + j

            @pl.when(t < nch)
            def _():
                lin_start(abase + t * _CHUNK, j)

            @pl.when(jnp.logical_and(t >= 1, t - 1 < nch))
            def _():
                lin_wait(abase + (t - 1) * _CHUNK, (j + 2) % 3)
                gat_start((j + 2) % 3)

            @pl.when(jnp.logical_and(t >= 2, t - 2 < nch))
            def _():
                gat_wait((j + 1) % 3)
                accum((j + 1) % 3, abase + (t - 2) * _CHUNK)

        return carry

    lax.fori_loop(0, (nch + 4) // 3, steps, 0)

    r0 = w * _RPW
    pltpu.sync_copy(acc.at[pl.ds(0, 128)], out_hbm.at[pl.ds(r0, 128)])
    pltpu.sync_copy(acc.at[pl.ds(128, 128)],
                    out_hbm.at[pl.ds(r0 + 128, 128)])
    pltpu.sync_copy(acc.at[pl.ds(256, 64)],
                    out_hbm.at[pl.ds(r0 + 256, 64)])


def _sc_edge(l, x, ee, src_s, dstloc, starts):
    return pl.kernel(
        functools.partial(_sc_body, l),
        out_type=jax.ShapeDtypeStruct((_NACC, D), jnp.float32),
        mesh=plsc.VectorSubcoreMesh(core_axis_name="c", subcore_axis_name="s",
                                    num_cores=_NCORE, num_subcores=_NSUB),
        scratch_types=[
            pltpu.VMEM((64,), jnp.int32),
            pltpu.VMEM((3, _CHUNK), jnp.int32),
            pltpu.VMEM((3, 1, 2 * _CHUNK), jnp.int32),
            pltpu.VMEM((3, _CHUNK, D), jnp.float32),
            pltpu.VMEM((_RPW, D), jnp.float32),
            pltpu.SemaphoreType.DMA((3,)),
            pltpu.SemaphoreType.DMA((3,)),
        ],
    )(x, ee, src_s, dstloc, starts)


_DL_BLK = 1000


def _dense_body(x_ref, a_ref, w1_ref, b1_ref, w2_ref, b2_ref,
                wih_ref, whh_ref, bih_ref, bhh_ref, o_ref):
    x = x_ref[...]
    t = x + a_ref[...]
    t = jnp.dot(t, w1_ref[...], preferred_element_type=jnp.float32) + b1_ref[...]
    t = _leaky(t * _BN)
    h = jnp.dot(t, w2_ref[...], preferred_element_type=jnp.float32) + b2_ref[...]
    h = jnp.where(h > 0, h, jnp.exp(h) - 1.0)
    gi = jnp.dot(h, wih_ref[...], preferred_element_type=jnp.float32) + bih_ref[...]
    gh = jnp.dot(x, whh_ref[...], preferred_element_type=jnp.float32) + bhh_ref[...]
    r = jax.nn.sigmoid(gi[:, :D] + gh[:, :D])
    z = jax.nn.sigmoid(gi[:, D:2 * D] + gh[:, D:2 * D])
    n = jnp.tanh(gi[:, 2 * D:] + r * gh[:, 2 * D:])
    o_ref[...] = _leaky((1.0 - z) * n + z * x)


def _dense_layer(x, aggr, w1, b1, w2, b2, wih, whh, bih, bhh):
    full = lambda s: pl.BlockSpec(s, lambda i: tuple(0 for _ in s))
    return pl.pallas_call(
        _dense_body,
        grid=(N // _DL_BLK,),
        in_specs=[
            pl.BlockSpec((_DL_BLK, D), lambda i: (i, 0)),
            pl.BlockSpec((_DL_BLK, D), lambda i: (i, 0)),
            full((D, D)), full((1, D)), full((D, D)), full((1, D)),
            full((D, 3 * D)), full((D, 3 * D)), full((1, 3 * D)), full((1, 3 * D)),
        ],
        out_specs=pl.BlockSpec((_DL_BLK, D), lambda i: (i, 0)),
        out_shape=jax.ShapeDtypeStruct((N, D), jnp.float32),
    )(x, aggr, w1, b1.reshape(1, D), w2, b2.reshape(1, D),
      wih, whh, bih.reshape(1, 3 * D), bhh.reshape(1, 3 * D))


def _readout_body(x_ref, b_ref, wl_ref, wr_ref, att_ref, bias_ref,
                  wih_ref, whh_ref, bih_ref, bhh_ref, lw_ref, lb_ref, o_ref):
    x = x_ref[...]
    oh = (b_ref[...] == lax.broadcasted_iota(jnp.int32, (N, G), 1)).astype(
        jnp.float32)
    dn = (((0,), (0,)), ((), ()))
    pool = lax.dot_general(oh, x, dn, preferred_element_type=jnp.float32)
    out = _leaky(pool)
    xl = jnp.dot(x, wl_ref[...], preferred_element_type=jnp.float32)
    att = att_ref[...]
    for _ in range(STEPS):
        xr = jnp.dot(out, wr_ref[...], preferred_element_type=jnp.float32)
        z = xl + jnp.dot(oh, xr, preferred_element_type=jnp.float32)
        z = jnp.where(z >= 0, z, 0.2 * z)
        e = jnp.sum(z * att, axis=1, keepdims=True)
        m = jnp.max(jnp.where(oh > 0, e, -jnp.inf), axis=0, keepdims=True)
        m = jnp.where(jnp.isfinite(m), m, 0.0)
        ex = jnp.exp(e - jnp.sum(oh * m, axis=1, keepdims=True))
        den = lax.dot_general(oh, ex, dn, preferred_element_type=jnp.float32)
        den_b = jnp.dot(oh, den, preferred_element_type=jnp.float32)
        alpha = ex / jnp.maximum(den_b, 1e-16)
        h = lax.dot_general(oh, alpha * xl, dn,
                            preferred_element_type=jnp.float32) + bias_ref[...]
        h = jnp.where(h > 0, h, jnp.exp(h) - 1.0)
        gi = jnp.dot(h, wih_ref[...], preferred_element_type=jnp.float32) \
            + bih_ref[...]
        gh = jnp.dot(out, whh_ref[...], preferred_element_type=jnp.float32) \
            + bhh_ref[...]
        r = jax.nn.sigmoid(gi[:, :D] + gh[:, :D])
        zz = jax.nn.sigmoid(gi[:, D:2 * D] + gh[:, D:2 * D])
        n = jnp.tanh(gi[:, 2 * D:] + r * gh[:, 2 * D:])
        out = _leaky((1.0 - zz) * n + zz * out)
    o_ref[...] = jnp.dot(out, lw_ref[...], preferred_element_type=jnp.float32) \
        + lb_ref[...]


def _readout(x, batch, mol_Wl, mol_Wr, mol_att, mol_bias,
             molgru_Wih, molgru_Whh, molgru_bih, molgru_bhh, lin_W, lin_b):
    full = lambda s: pl.BlockSpec(s, lambda: tuple(0 for _ in s))
    return pl.pallas_call(
        _readout_body,
        in_specs=[
            full((N, D)), full((N, 1)),
            full((D, D)), full((D, D)), full((1, D)), full((1, D)),
            full((D, 3 * D)), full((D, 3 * D)), full((1, 3 * D)),
            full((1, 3 * D)), full((D, D)), full((1, D)),
        ],
        out_specs=full((G, D)),
        out_shape=jax.ShapeDtypeStruct((G, D), jnp.float32),
    )(x, batch.reshape(N, 1), mol_Wl, mol_Wr, mol_att.reshape(1, D),
      mol_bias.reshape(1, D), molgru_Wih, molgru_Whh,
      molgru_bih.reshape(1, 3 * D), molgru_bhh.reshape(1, 3 * D),
      lin_W, lin_b.reshape(1, D))


def kernel(x, edge_index, edge_attr, batch, We, be, lin1_W, lin1_b, lin2_W,
           lin2_b, gru_Wih, gru_Whh, gru_bih, gru_bhh, mol_Wl, mol_Wr,
           mol_att, mol_bias, molgru_Wih, molgru_Whh, molgru_bih, molgru_bhh,
           lin_W, lin_b):
    perm = jnp.argsort(edge_index[1])
    dst_s = edge_index[1][perm]
    src_s = jnp.concatenate(
        [edge_index[0][perm], jnp.zeros((_EP - E,), jnp.int32)])
    dstloc = jnp.concatenate(
        [dst_s % _RPW, jnp.zeros((_EP - E,), jnp.int32)])
    ea_s = jnp.concatenate(
        [edge_attr[perm], jnp.zeros((_EP - E, ED), jnp.float32)])
    bounds = jnp.searchsorted(
        dst_s, jnp.arange(_NW + 1, dtype=jnp.int32) * _RPW).astype(jnp.int32)
    starts = jnp.zeros((64,), jnp.int32).at[:_NW + 1].set(bounds)

    e_emb3 = _edge_emb(ea_s, We, be.reshape(L, 1, D))
    for l in range(L):
        aggr = _sc_edge(l, x, e_emb3, src_s, dstloc, starts)
        x = _dense_layer(x, aggr, lin1_W[l], lin1_b[l], lin2_W[l],
                         lin2_b[l], gru_Wih[l], gru_Whh[l], gru_bih[l],
                         gru_bhh[l])
    return _readout(x, batch, mol_Wl, mol_Wr, mol_att, mol_bias,
                    molgru_Wih, molgru_Whh, molgru_bih, molgru_bhh,
                    lin_W, lin_b)


# speedup vs baseline: 1.0790x; 1.0790x over previous
import functools
import math

import jax
import jax.numpy as jnp
from jax import lax
from jax.experimental import pallas as pl
from jax.experimental.pallas import tpu as pltpu
from jax.experimental.pallas import tpu_sc as plsc

N = 10000
E = 320000
D = 128
ED = 16
L = 3
G = 128
STEPS = 2

_NCORE = 2
_NSUB = 16
_NW = _NCORE * _NSUB
_CHUNK = 128
_RPW = 320
_NACC = _RPW * _NW
_EP = 327680

_BN = 1.0 / math.sqrt(1.0 + 1e-5)


def _leaky(v, s=0.01):
    return jnp.where(v >= 0, v, s * v)


_EE_BLK = 8000


def _ee_body(ea_ref, w_ref, b_ref, out_ref):
    out_ref[0] = (
        jnp.dot(ea_ref[...], w_ref[0], preferred_element_type=jnp.float32)
        + b_ref[0]
    )


def _edge_emb(edge_attr, We, be3):
    return pl.pallas_call(
        _ee_body,
        grid=(L, E // _EE_BLK),
        in_specs=[
            pl.BlockSpec((_EE_BLK, ED), lambda l, i: (i, 0)),
            pl.BlockSpec((1, ED, D), lambda l, i: (l, 0, 0)),
            pl.BlockSpec((1, 1, D), lambda l, i: (l, 0, 0)),
        ],
        out_specs=pl.BlockSpec((1, _EE_BLK, D), lambda l, i: (l, i, 0)),
        out_shape=jax.ShapeDtypeStruct((L, E, D), jnp.float32),
    )(edge_attr, We, be3)


def _sc_body(l, x_hbm, ee_hbm, src_hbm, perm_hbm, dstloc_hbm, starts_hbm,
             out_hbm, stv, pidxv, srcv, dstv, buf, acc,
             sem_lin, sem_iga, sem_gat):
    c = lax.axis_index("c")
    s = lax.axis_index("s")
    w = c * _NSUB + s

    pltpu.sync_copy(starts_hbm, stv)
    win = stv[pl.ds(w, 16)]
    start = win[0]
    end = win[1]
    abase = (start // 8) * 8
    nch = (end - abase + _CHUNK - 1) // _CHUNK

    zero16 = jnp.zeros((16,), jnp.float32)

    def zrow(r, carry):
        for k in range(8):
            acc[r, pl.ds(k * 16, 16)] = zero16
        return carry

    lax.fori_loop(0, _RPW, zrow, 0)

    def lin_start(cb, b):
        pltpu.async_copy(perm_hbm.at[pl.ds(cb, _CHUNK)],
                         pidxv.at[b], sem_lin.at[b])
        pltpu.async_copy(dstloc_hbm.at[pl.ds(cb, 2 * _CHUNK)],
                         dstv.at[b, 0], sem_lin.at[b])

    def lin_wait(cb, b):
        pltpu.make_async_copy(perm_hbm.at[pl.ds(cb, _CHUNK)],
                              pidxv.at[b], sem_lin.at[b]).wait()
        pltpu.make_async_copy(dstloc_hbm.at[pl.ds(cb, 2 * _CHUNK)],
                              dstv.at[b, 0], sem_lin.at[b]).wait()

    def iga_start(b):
        pltpu.async_copy(src_hbm.at[pidxv.at[b]], srcv.at[b], sem_iga.at[b])
        pltpu.async_copy(ee_hbm.at[l].at[pidxv.at[b]], buf.at[b],
                         sem_iga.at[b])

    def iga_wait(b):
        pltpu.make_async_copy(src_hbm.at[pidxv.at[b]], srcv.at[b],
                              sem_iga.at[b]).wait()
        pltpu.make_async_copy(ee_hbm.at[l].at[pidxv.at[b]], buf.at[b],
                              sem_iga.at[b]).wait()

    def gat_start(b):
        pltpu.async_copy(x_hbm.at[srcv.at[b]], buf.at[b], sem_gat.at[b],
                         add=True)

    def gat_wait(b):
        pltpu.make_async_copy(x_hbm.at[srcv.at[b]], buf.at[b],
                              sem_gat.at[b]).wait()

    def accum(b, cb):
        e_lo = jnp.maximum(start - cb, 0)
        e_hi = jnp.minimum(end - cb, _CHUNK)
        full = jnp.logical_and(e_lo == 0, e_hi == _CHUNK)

        @pl.when(full)
        def _():
            def group(g, carry):
                base = g * 16
                wv = dstv[b, 0, pl.ds(base, 16)]
                for i in range(16):
                    row = wv[i]
                    for k in range(8):
                        val = jnp.maximum(
                            buf[b, base + i, pl.ds(k * 16, 16)], 0.0)
                        plsc.addupdate(acc.at[row, pl.ds(k * 16, 16)], val)
                return carry

            lax.fori_loop(0, 8, group, 0)

        @pl.when(jnp.logical_not(full))
        def _():
            def per_edge(e, carry):
                wv = dstv[b, 0, pl.ds(e, 16)]
                row = wv[0]
                for k in range(8):
                    val = jnp.maximum(buf[b, e, pl.ds(k * 16, 16)], 0.0)
                    plsc.addupdate(acc.at[row, pl.ds(k * 16, 16)], val)
                return carry

            lax.fori_loop(e_lo, e_hi, per_edge, 0)

    def steps(to, carry):
        for j in range(3):
            t = 3 * to + j

            @pl.when(jnp.logical_and(t >= 3, t - 3 < nch))
            def _():
                gat_wait(j)
                accum(j, abase + (t - 3) * _CHUNK)

            @pl.when(t < nch)
            def _():
                lin_start(abase + t * _CHUNK, j)

            @pl.when(jnp.logical_and(t >= 1, t - 1 < nch))
            def _():
                lin_wait(abase + (t - 1) * _CHUNK, (j + 2) % 3)
                iga_start((j + 2) % 3)

            @pl.when(jnp.logical_and(t >= 2, t - 2 < nch))
            def _():
                iga_wait((j + 1) % 3)
                gat_start((j + 1) % 3)

        return carry

    lax.fori_loop(0, (nch + 5) // 3, steps, 0)

    r0 = w * _RPW
    pltpu.sync_copy(acc.at[pl.ds(0, 128)], out_hbm.at[pl.ds(r0, 128)])
    pltpu.sync_copy(acc.at[pl.ds(128, 128)],
                    out_hbm.at[pl.ds(r0 + 128, 128)])
    pltpu.sync_copy(acc.at[pl.ds(256, 64)],
                    out_hbm.at[pl.ds(r0 + 256, 64)])


def _sc_edge(l, x, ee, src, perm_p, dstloc_p, starts):
    return pl.kernel(
        functools.partial(_sc_body, l),
        out_type=jax.ShapeDtypeStruct((_NACC, D), jnp.float32),
        mesh=plsc.VectorSubcoreMesh(core_axis_name="c", subcore_axis_name="s",
                                    num_cores=_NCORE, num_subcores=_NSUB),
        scratch_types=[
            pltpu.VMEM((64,), jnp.int32),
            pltpu.VMEM((3, _CHUNK), jnp.int32),
            pltpu.VMEM((3, _CHUNK), jnp.int32),
            pltpu.VMEM((3, 1, 2 * _CHUNK), jnp.int32),
            pltpu.VMEM((3, _CHUNK, D), jnp.float32),
            pltpu.VMEM((_RPW, D), jnp.float32),
            pltpu.SemaphoreType.DMA((3,)),
            pltpu.SemaphoreType.DMA((3,)),
            pltpu.SemaphoreType.DMA((3,)),
        ],
    )(x, ee, src, perm_p, dstloc_p, starts)


_DL_BLK = 1000


def _dense_body(x_ref, a_ref, w1_ref, b1_ref, w2_ref, b2_ref,
                wih_ref, whh_ref, bih_ref, bhh_ref, o_ref):
    x = x_ref[...]
    t = x + a_ref[...]
    t = jnp.dot(t, w1_ref[...], preferred_element_type=jnp.float32) + b1_ref[...]
    t = _leaky(t * _BN)
    h = jnp.dot(t, w2_ref[...], preferred_element_type=jnp.float32) + b2_ref[...]
    h = jnp.where(h > 0, h, jnp.exp(h) - 1.0)
    gi = jnp.dot(h, wih_ref[...], preferred_element_type=jnp.float32) + bih_ref[...]
    gh = jnp.dot(x, whh_ref[...], preferred_element_type=jnp.float32) + bhh_ref[...]
    r = jax.nn.sigmoid(gi[:, :D] + gh[:, :D])
    z = jax.nn.sigmoid(gi[:, D:2 * D] + gh[:, D:2 * D])
    n = jnp.tanh(gi[:, 2 * D:] + r * gh[:, 2 * D:])
    o_ref[...] = _leaky((1.0 - z) * n + z * x)


def _dense_layer(x, aggr, w1, b1, w2, b2, wih, whh, bih, bhh):
    full = lambda s: pl.BlockSpec(s, lambda i: tuple(0 for _ in s))
    return pl.pallas_call(
        _dense_body,
        grid=(N // _DL_BLK,),
        in_specs=[
            pl.BlockSpec((_DL_BLK, D), lambda i: (i, 0)),
            pl.BlockSpec((_DL_BLK, D), lambda i: (i, 0)),
            full((D, D)), full((1, D)), full((D, D)), full((1, D)),
            full((D, 3 * D)), full((D, 3 * D)), full((1, 3 * D)), full((1, 3 * D)),
        ],
        out_specs=pl.BlockSpec((_DL_BLK, D), lambda i: (i, 0)),
        out_shape=jax.ShapeDtypeStruct((N, D), jnp.float32),
    )(x, aggr, w1, b1.reshape(1, D), w2, b2.reshape(1, D),
      wih, whh, bih.reshape(1, 3 * D), bhh.reshape(1, 3 * D))


def _readout_body(x_ref, b_ref, wl_ref, wr_ref, att_ref, bias_ref,
                  wih_ref, whh_ref, bih_ref, bhh_ref, lw_ref, lb_ref, o_ref):
    x = x_ref[...]
    oh = (b_ref[...] == lax.broadcasted_iota(jnp.int32, (N, G), 1)).astype(
        jnp.float32)
    dn = (((0,), (0,)), ((), ()))
    pool = lax.dot_general(oh, x, dn, preferred_element_type=jnp.float32)
    out = _leaky(pool)
    xl = jnp.dot(x, wl_ref[...], preferred_element_type=jnp.float32)
    att = att_ref[...]
    for _ in range(STEPS):
        xr = jnp.dot(out, wr_ref[...], preferred_element_type=jnp.float32)
        z = xl + jnp.dot(oh, xr, preferred_element_type=jnp.float32)
        z = jnp.where(z >= 0, z, 0.2 * z)
        e = jnp.sum(z * att, axis=1, keepdims=True)
        m = jnp.max(jnp.where(oh > 0, e, -jnp.inf), axis=0, keepdims=True)
        m = jnp.where(jnp.isfinite(m), m, 0.0)
        ex = jnp.exp(e - jnp.sum(oh * m, axis=1, keepdims=True))
        den = lax.dot_general(oh, ex, dn, preferred_element_type=jnp.float32)
        den_b = jnp.dot(oh, den, preferred_element_type=jnp.float32)
        alpha = ex / jnp.maximum(den_b, 1e-16)
        h = lax.dot_general(oh, alpha * xl, dn,
                            preferred_element_type=jnp.float32) + bias_ref[...]
        h = jnp.where(h > 0, h, jnp.exp(h) - 1.0)
        gi = jnp.dot(h, wih_ref[...], preferred_element_type=jnp.float32) \
            + bih_ref[...]
        gh = jnp.dot(out, whh_ref[...], preferred_element_type=jnp.float32) \
            + bhh_ref[...]
        r = jax.nn.sigmoid(gi[:, :D] + gh[:, :D])
        zz = jax.nn.sigmoid(gi[:, D:2 * D] + gh[:, D:2 * D])
        n = jnp.tanh(gi[:, 2 * D:] + r * gh[:, 2 * D:])
        out = _leaky((1.0 - zz) * n + zz * out)
    o_ref[...] = jnp.dot(out, lw_ref[...], preferred_element_type=jnp.float32) \
        + lb_ref[...]


def _readout(x, batch, mol_Wl, mol_Wr, mol_att, mol_bias,
             molgru_Wih, molgru_Whh, molgru_bih, molgru_bhh, lin_W, lin_b):
    full = lambda s: pl.BlockSpec(s, lambda: tuple(0 for _ in s))
    return pl.pallas_call(
        _readout_body,
        in_specs=[
            full((N, D)), full((N, 1)),
            full((D, D)), full((D, D)), full((1, D)), full((1, D)),
            full((D, 3 * D)), full((D, 3 * D)), full((1, 3 * D)),
            full((1, 3 * D)), full((D, D)), full((1, D)),
        ],
        out_specs=full((G, D)),
        out_shape=jax.ShapeDtypeStruct((G, D), jnp.float32),
    )(x, batch.reshape(N, 1), mol_Wl, mol_Wr, mol_att.reshape(1, D),
      mol_bias.reshape(1, D), molgru_Wih, molgru_Whh,
      molgru_bih.reshape(1, 3 * D), molgru_bhh.reshape(1, 3 * D),
      lin_W, lin_b.reshape(1, D))


def kernel(x, edge_index, edge_attr, batch, We, be, lin1_W, lin1_b, lin2_W,
           lin2_b, gru_Wih, gru_Whh, gru_bih, gru_bhh, mol_Wl, mol_Wr,
           mol_att, mol_bias, molgru_Wih, molgru_Whh, molgru_bih, molgru_bhh,
           lin_W, lin_b):
    dst_s, perm = lax.sort_key_val(edge_index[1],
                                   lax.iota(jnp.int32, E))
    perm_p = jnp.concatenate([perm, jnp.zeros((_EP - E,), jnp.int32)])
    dstloc_p = jnp.concatenate(
        [dst_s % _RPW, jnp.zeros((_EP - E,), jnp.int32)])
    bounds = jnp.searchsorted(
        dst_s, jnp.arange(_NW + 1, dtype=jnp.int32) * _RPW).astype(jnp.int32)
    starts = jnp.zeros((64,), jnp.int32).at[:_NW + 1].set(bounds)

    e_emb3 = _edge_emb(edge_attr, We, be.reshape(L, 1, D))
    for l in range(L):
        aggr = _sc_edge(l, x, e_emb3, edge_index[0], perm_p,
                        dstloc_p, starts)
        x = _dense_layer(x, aggr, lin1_W[l], lin1_b[l], lin2_W[l],
                         lin2_b[l], gru_Wih[l], gru_Whh[l], gru_bih[l],
                         gru_bhh[l])
    return _readout(x, batch, mol_Wl, mol_Wr, mol_att, mol_bias,
                    molgru_Wih, molgru_Whh, molgru_bih, molgru_bhh,
                    lin_W, lin_b)


# speedup vs baseline: 1.1611x; 1.0761x over previous
import functools
import math

import jax
import jax.numpy as jnp
from jax import lax
from jax.experimental import pallas as pl
from jax.experimental.pallas import tpu as pltpu
from jax.experimental.pallas import tpu_sc as plsc

N = 10000
E = 320000
D = 128
ED = 16
L = 3
G = 128
STEPS = 2

_NCORE = 2
_NSUB = 16
_NW = _NCORE * _NSUB
_CHUNK = 128
_RPW = 320
_NACC = _RPW * _NW
_EP = 327680

_BN = 1.0 / math.sqrt(1.0 + 1e-5)


def _leaky(v, s=0.01):
    return jnp.where(v >= 0, v, s * v)


_EE_BLK = 8000


def _ee_body(ea_ref, w_ref, b_ref, out_ref):
    out_ref[...] = (
        jnp.dot(ea_ref[...], w_ref[...], preferred_element_type=jnp.float32)
        + b_ref[...]
    )


def _edge_emb(edge_attr, We_l, be_l):
    full = lambda sh: pl.BlockSpec(sh, lambda i: tuple(0 for _ in sh))
    return pl.pallas_call(
        _ee_body,
        grid=(E // _EE_BLK,),
        in_specs=[
            pl.BlockSpec((_EE_BLK, ED), lambda i: (i, 0)),
            full((ED, D)),
            full((1, D)),
        ],
        out_specs=pl.BlockSpec((_EE_BLK, D), lambda i: (i, 0)),
        out_shape=jax.ShapeDtypeStruct((E, D), jnp.float32),
    )(edge_attr, We_l, be_l)


def _sc_body(l, x_hbm, ee_hbm, src_hbm, perm_hbm, dstloc_hbm, starts_hbm,
             out_hbm, stv, pidxv, srcv, dstv, buf, acc,
             sem_lin, sem_iga, sem_gat):
    c = lax.axis_index("c")
    s = lax.axis_index("s")
    w = c * _NSUB + s

    pltpu.sync_copy(starts_hbm, stv)
    win = stv[pl.ds(w, 16)]
    start = win[0]
    end = win[1]
    abase = (start // 8) * 8
    nch = (end - abase + _CHUNK - 1) // _CHUNK

    zero16 = jnp.zeros((16,), jnp.float32)

    def zrow(r, carry):
        for k in range(8):
            acc[r, pl.ds(k * 16, 16)] = zero16
        return carry

    lax.fori_loop(0, _RPW, zrow, 0)

    def lin_start(cb, b):
        pltpu.async_copy(perm_hbm.at[pl.ds(cb, _CHUNK)],
                         pidxv.at[b], sem_lin.at[b])
        pltpu.async_copy(dstloc_hbm.at[pl.ds(cb, 2 * _CHUNK)],
                         dstv.at[b, 0], sem_lin.at[b])

    def lin_wait(cb, b):
        pltpu.make_async_copy(perm_hbm.at[pl.ds(cb, _CHUNK)],
                              pidxv.at[b], sem_lin.at[b]).wait()
        pltpu.make_async_copy(dstloc_hbm.at[pl.ds(cb, 2 * _CHUNK)],
                              dstv.at[b, 0], sem_lin.at[b]).wait()

    def iga_start(b):
        pltpu.async_copy(src_hbm.at[pidxv.at[b]], srcv.at[b], sem_iga.at[b])
        pltpu.async_copy(ee_hbm.at[pidxv.at[b]], buf.at[b], sem_iga.at[b])

    def iga_wait(b):
        pltpu.make_async_copy(src_hbm.at[pidxv.at[b]], srcv.at[b],
                              sem_iga.at[b]).wait()
        pltpu.make_async_copy(ee_hbm.at[pidxv.at[b]], buf.at[b],
                              sem_iga.at[b]).wait()

    def gat_start(b):
        pltpu.async_copy(x_hbm.at[srcv.at[b]], buf.at[b], sem_gat.at[b],
                         add=True)

    def gat_wait(b):
        pltpu.make_async_copy(x_hbm.at[srcv.at[b]], buf.at[b],
                              sem_gat.at[b]).wait()

    def accum(b, cb):
        e_lo = jnp.maximum(start - cb, 0)
        e_hi = jnp.minimum(end - cb, _CHUNK)
        full = jnp.logical_and(e_lo == 0, e_hi == _CHUNK)

        @pl.when(full)
        def _():
            def group(g, carry):
                base = g * 16
                wv = dstv[b, 0, pl.ds(base, 16)]
                for i in range(16):
                    row = wv[i]
                    for k in range(8):
                        val = jnp.maximum(
                            buf[b, base + i, pl.ds(k * 16, 16)], 0.0)
                        plsc.addupdate(acc.at[row, pl.ds(k * 16, 16)], val)
                return carry

            lax.fori_loop(0, 8, group, 0)

        @pl.when(jnp.logical_not(full))
        def _():
            def per_edge(e, carry):
                wv = dstv[b, 0, pl.ds(e, 16)]
                row = wv[0]
                for k in range(8):
                    val = jnp.maximum(buf[b, e, pl.ds(k * 16, 16)], 0.0)
                    plsc.addupdate(acc.at[row, pl.ds(k * 16, 16)], val)
                return carry

            lax.fori_loop(e_lo, e_hi, per_edge, 0)

    def steps(to, carry):
        for j in range(3):
            t = 3 * to + j

            @pl.when(jnp.logical_and(t >= 3, t - 3 < nch))
            def _():
                gat_wait(j)
                accum(j, abase + (t - 3) * _CHUNK)

            @pl.when(t < nch)
            def _():
                lin_start(abase + t * _CHUNK, j)

            @pl.when(jnp.logical_and(t >= 1, t - 1 < nch))
            def _():
                lin_wait(abase + (t - 1) * _CHUNK, (j + 2) % 3)
                iga_start((j + 2) % 3)

            @pl.when(jnp.logical_and(t >= 2, t - 2 < nch))
            def _():
                iga_wait((j + 1) % 3)
                gat_start((j + 1) % 3)

        return carry

    lax.fori_loop(0, (nch + 5) // 3, steps, 0)

    r0 = w * _RPW
    pltpu.sync_copy(acc.at[pl.ds(0, 128)], out_hbm.at[pl.ds(r0, 128)])
    pltpu.sync_copy(acc.at[pl.ds(128, 128)],
                    out_hbm.at[pl.ds(r0 + 128, 128)])
    pltpu.sync_copy(acc.at[pl.ds(256, 64)],
                    out_hbm.at[pl.ds(r0 + 256, 64)])


def _sc_edge(l, x, ee, src, perm_p, dstloc_p, starts):
    return pl.kernel(
        functools.partial(_sc_body, l),
        out_type=jax.ShapeDtypeStruct((_NACC, D), jnp.float32),
        mesh=plsc.VectorSubcoreMesh(core_axis_name="c", subcore_axis_name="s",
                                    num_cores=_NCORE, num_subcores=_NSUB),
        scratch_types=[
            pltpu.VMEM((64,), jnp.int32),
            pltpu.VMEM((3, _CHUNK), jnp.int32),
            pltpu.VMEM((3, _CHUNK), jnp.int32),
            pltpu.VMEM((3, 1, 2 * _CHUNK), jnp.int32),
            pltpu.VMEM((3, _CHUNK, D), jnp.float32),
            pltpu.VMEM((_RPW, D), jnp.float32),
            pltpu.SemaphoreType.DMA((3,)),
            pltpu.SemaphoreType.DMA((3,)),
            pltpu.SemaphoreType.DMA((3,)),
        ],
    )(x, ee, src, perm_p, dstloc_p, starts)


_DL_BLK = 1000


def _dense_body(x_ref, a_ref, w1_ref, b1_ref, w2_ref, b2_ref,
                wih_ref, whh_ref, bih_ref, bhh_ref, o_ref):
    x = x_ref[...]
    t = x + a_ref[...]
    t = jnp.dot(t, w1_ref[...], preferred_element_type=jnp.float32) + b1_ref[...]
    t = _leaky(t * _BN)
    h = jnp.dot(t, w2_ref[...], preferred_element_type=jnp.float32) + b2_ref[...]
    h = jnp.where(h > 0, h, jnp.exp(h) - 1.0)
    gi = jnp.dot(h, wih_ref[...], preferred_element_type=jnp.float32) + bih_ref[...]
    gh = jnp.dot(x, whh_ref[...], preferred_element_type=jnp.float32) + bhh_ref[...]
    r = jax.nn.sigmoid(gi[:, :D] + gh[:, :D])
    z = jax.nn.sigmoid(gi[:, D:2 * D] + gh[:, D:2 * D])
    n = jnp.tanh(gi[:, 2 * D:] + r * gh[:, 2 * D:])
    o_ref[...] = _leaky((1.0 - z) * n + z * x)


def _dense_layer(x, aggr, w1, b1, w2, b2, wih, whh, bih, bhh):
    full = lambda s: pl.BlockSpec(s, lambda i: tuple(0 for _ in s))
    return pl.pallas_call(
        _dense_body,
        grid=(N // _DL_BLK,),
        in_specs=[
            pl.BlockSpec((_DL_BLK, D), lambda i: (i, 0)),
            pl.BlockSpec((_DL_BLK, D), lambda i: (i, 0)),
            full((D, D)), full((1, D)), full((D, D)), full((1, D)),
            full((D, 3 * D)), full((D, 3 * D)), full((1, 3 * D)), full((1, 3 * D)),
        ],
        out_specs=pl.BlockSpec((_DL_BLK, D), lambda i: (i, 0)),
        out_shape=jax.ShapeDtypeStruct((N, D), jnp.float32),
    )(x, aggr, w1, b1.reshape(1, D), w2, b2.reshape(1, D),
      wih, whh, bih.reshape(1, 3 * D), bhh.reshape(1, 3 * D))


def _readout_body(x_ref, b_ref, wl_ref, wr_ref, att_ref, bias_ref,
                  wih_ref, whh_ref, bih_ref, bhh_ref, lw_ref, lb_ref, o_ref):
    x = x_ref[...]
    oh = (b_ref[...] == lax.broadcasted_iota(jnp.int32, (N, G), 1)).astype(
        jnp.float32)
    dn = (((0,), (0,)), ((), ()))
    pool = lax.dot_general(oh, x, dn, preferred_element_type=jnp.float32)
    out = _leaky(pool)
    xl = jnp.dot(x, wl_ref[...], preferred_element_type=jnp.float32)
    att = att_ref[...]
    for _ in range(STEPS):
        xr = jnp.dot(out, wr_ref[...], preferred_element_type=jnp.float32)
        z = xl + jnp.dot(oh, xr, preferred_element_type=jnp.float32)
        z = jnp.where(z >= 0, z, 0.2 * z)
        e = jnp.sum(z * att, axis=1, keepdims=True)
        m = jnp.max(jnp.where(oh > 0, e, -jnp.inf), axis=0, keepdims=True)
        m = jnp.where(jnp.isfinite(m), m, 0.0)
        ex = jnp.exp(e - jnp.sum(oh * m, axis=1, keepdims=True))
        den = lax.dot_general(oh, ex, dn, preferred_element_type=jnp.float32)
        den_b = jnp.dot(oh, den, preferred_element_type=jnp.float32)
        alpha = ex / jnp.maximum(den_b, 1e-16)
        h = lax.dot_general(oh, alpha * xl, dn,
                            preferred_element_type=jnp.float32) + bias_ref[...]
        h = jnp.where(h > 0, h, jnp.exp(h) - 1.0)
        gi = jnp.dot(h, wih_ref[...], preferred_element_type=jnp.float32) \
            + bih_ref[...]
        gh = jnp.dot(out, whh_ref[...], preferred_element_type=jnp.float32) \
            + bhh_ref[...]
        r = jax.nn.sigmoid(gi[:, :D] + gh[:, :D])
        zz = jax.nn.sigmoid(gi[:, D:2 * D] + gh[:, D:2 * D])
        n = jnp.tanh(gi[:, 2 * D:] + r * gh[:, 2 * D:])
        out = _leaky((1.0 - zz) * n + zz * out)
    o_ref[...] = jnp.dot(out, lw_ref[...], preferred_element_type=jnp.float32) \
        + lb_ref[...]


def _readout(x, batch, mol_Wl, mol_Wr, mol_att, mol_bias,
             molgru_Wih, molgru_Whh, molgru_bih, molgru_bhh, lin_W, lin_b):
    full = lambda s: pl.BlockSpec(s, lambda: tuple(0 for _ in s))
    return pl.pallas_call(
        _readout_body,
        in_specs=[
            full((N, D)), full((N, 1)),
            full((D, D)), full((D, D)), full((1, D)), full((1, D)),
            full((D, 3 * D)), full((D, 3 * D)), full((1, 3 * D)),
            full((1, 3 * D)), full((D, D)), full((1, D)),
        ],
        out_specs=full((G, D)),
        out_shape=jax.ShapeDtypeStruct((G, D), jnp.float32),
    )(x, batch.reshape(N, 1), mol_Wl, mol_Wr, mol_att.reshape(1, D),
      mol_bias.reshape(1, D), molgru_Wih, molgru_Whh,
      molgru_bih.reshape(1, 3 * D), molgru_bhh.reshape(1, 3 * D),
      lin_W, lin_b.reshape(1, D))


def kernel(x, edge_index, edge_attr, batch, We, be, lin1_W, lin1_b, lin2_W,
           lin2_b, gru_Wih, gru_Whh, gru_bih, gru_bhh, mol_Wl, mol_Wr,
           mol_att, mol_bias, molgru_Wih, molgru_Whh, molgru_bih, molgru_bhh,
           lin_W, lin_b):
    dst_s, perm = lax.sort_key_val(edge_index[1],
                                   lax.iota(jnp.int32, E))
    perm_p = jnp.concatenate([perm, jnp.zeros((_EP - E,), jnp.int32)])
    dstloc_p = jnp.concatenate(
        [dst_s % _RPW, jnp.zeros((_EP - E,), jnp.int32)])
    bounds = jnp.searchsorted(
        dst_s, jnp.arange(_NW + 1, dtype=jnp.int32) * _RPW).astype(jnp.int32)
    starts = jnp.zeros((64,), jnp.int32).at[:_NW + 1].set(bounds)

    ee = [_edge_emb(edge_attr, We[l], be[l].reshape(1, D))
          for l in range(L)]
    for l in range(L):
        aggr = _sc_edge(l, x, ee[l], edge_index[0],
                        perm_p, dstloc_p, starts)
        x = _dense_layer(x, aggr, lin1_W[l], lin1_b[l], lin2_W[l],
                         lin2_b[l], gru_Wih[l], gru_Whh[l], gru_bih[l],
                         gru_bhh[l])
    return _readout(x, batch, mol_Wl, mol_Wr, mol_att, mol_bias,
                    molgru_Wih, molgru_Whh, molgru_bih, molgru_bhh,
                    lin_W, lin_b)
